# baseline (speedup 1.0000x reference)
import functools

import jax
import jax.numpy as jnp
from jax import lax
from jax.experimental import pallas as pl
from jax.experimental.pallas import tpu as pltpu
from jax.experimental.pallas import tpu_sc as plsc

_LANES = 16
_CHUNK = 64


def _repack(tab):
    n, d = tab.shape
    br = 2000
    nblk = n // 2 // br

    def body(a_ref, b_ref, out_ref):
        out_ref[...] = jnp.concatenate([a_ref[...], b_ref[...]], axis=1)

    return pl.pallas_call(
        body,
        grid=(nblk,),
        in_specs=[
            pl.BlockSpec((br, d), lambda i: (i, 0)),
            pl.BlockSpec((br, d), lambda i: (i + nblk, 0)),
        ],
        out_specs=pl.BlockSpec((br, 2 * d), lambda i: (i, 0)),
        out_shape=jax.ShapeDtypeStruct((n // 2, 2 * d), jnp.float32),
    )(tab, tab)


def kernel(user_ids, item_ids, user_table, item_table):
    batch = user_ids.shape[0]
    nrows, dim = user_table.shape
    pair = 128 // dim
    utab2 = _repack(user_table)
    itab2 = _repack(item_table)

    info = plsc.get_sparse_core_info()
    num_cores, num_subcores = info.num_cores, info.num_subcores
    num_workers = num_cores * num_subcores
    bpw = batch // num_workers
    nch = bpw // _CHUNK
    assert nch % 2 == 0

    mesh = plsc.VectorSubcoreMesh(core_axis_name="c", subcore_axis_name="s")

    @functools.partial(
        pl.kernel,
        out_type=jax.ShapeDtypeStruct((batch,), jnp.float32),
        mesh=mesh,
        scratch_types=[
            pltpu.VMEM((bpw,), jnp.int32),
            pltpu.VMEM((bpw,), jnp.int32),
            pltpu.VMEM((bpw,), jnp.int32),
            pltpu.VMEM((bpw,), jnp.int32),
            pltpu.VMEM((2, _CHUNK, 128), jnp.float32),
            pltpu.VMEM((2, _CHUNK, 128), jnp.float32),
            pltpu.VMEM((bpw,), jnp.float32),
            pltpu.VMEM((_LANES * (_LANES + 1),), jnp.float32),
            pltpu.SemaphoreType.DMA,
            pltpu.SemaphoreType.DMA,
        ],
        compiler_params=pltpu.CompilerParams(needs_layout_passes=False),
    )
    def mf(uids_hbm, iids_hbm, utab_hbm, itab_hbm, out_hbm,
           uidx_v, iidx_v, upair_v, ipair_v, urows_v, irows_v, out_v, tr_v,
           sems0, sems1):
        sems = (sems0, sems1)
        wid = lax.axis_index("s") * num_cores + lax.axis_index("c")
        base = wid * bpw
        pltpu.sync_copy(uids_hbm.at[pl.ds(base, bpw)], uidx_v)
        pltpu.sync_copy(iids_hbm.at[pl.ds(base, bpw)], iidx_v)

        half_rows = nrows // 2

        def pairs_body(k, carry):
            sl = pl.ds(k * _LANES, _LANES)
            uv = uidx_v[sl]
            iv = iidx_v[sl]
            upair_v[sl] = jnp.where(uv >= half_rows, uv - half_rows, uv)
            ipair_v[sl] = jnp.where(iv >= half_rows, iv - half_rows, iv)
            return carry

        lax.fori_loop(0, bpw // _LANES, pairs_body, 0)

        def fire(c, buf, sem):
            sl = pl.ds(c * _CHUNK, _CHUNK)
            pltpu.async_copy(utab_hbm.at[upair_v.at[sl]], urows_v.at[buf], sem)
            pltpu.async_copy(itab_hbm.at[ipair_v.at[sl]], irows_v.at[buf], sem)

        def wait_chunk(sem):
            pltpu.make_async_copy(
                utab_hbm.at[pl.ds(0, _CHUNK)], urows_v.at[0], sem).wait()
            pltpu.make_async_copy(
                itab_hbm.at[pl.ds(0, _CHUNK)], irows_v.at[0], sem).wait()

        lane_iota = lax.iota(jnp.int32, _LANES)
        tr_idx_base = lane_iota * (_LANES + 1)

        def compute(c, buf):
            for gg in range(_CHUNK // _LANES):
                uvec = uidx_v[pl.ds(c * _CHUNK + gg * _LANES, _LANES)]
                ivec = iidx_v[pl.ds(c * _CHUNK + gg * _LANES, _LANES)]
                for rr in range(_LANES):
                    j = gg * _LANES + rr
                    uhalf = jnp.where(uvec[rr] >= half_rows, dim, 0)
                    ihalf = jnp.where(ivec[rr] >= half_rows, dim, 0)
                    acc = None
                    for c4 in range(dim // _LANES):
                        u = urows_v[buf, j,
                                    pl.ds(uhalf + c4 * _LANES, _LANES)]
                        v = irows_v[buf, j,
                                    pl.ds(ihalf + c4 * _LANES, _LANES)]
                        p = u * v
                        acc = p if acc is None else acc + p
                    plsc.store_scatter(tr_v, [tr_idx_base + rr], acc)
                res = None
                for cc in range(_LANES):
                    col = tr_v[pl.ds(cc * (_LANES + 1), _LANES)]
                    res = col if res is None else res + col
                out_v[pl.ds(c * _CHUNK + gg * _LANES, _LANES)] = res

        fire(0, 0, sems[0])

        def body(c2, carry):
            c = 2 * c2
            fire(c + 1, 1, sems[1])
            wait_chunk(sems[0])
            compute(c, 0)

            @pl.when(c + 2 < nch)
            def _():
                fire(c + 2, 0, sems[0])

            wait_chunk(sems[1])
            compute(c + 1, 1)
            return carry

        lax.fori_loop(0, nch // 2, body, 0)
        pltpu.sync_copy(out_v, out_hbm.at[pl.ds(base, bpw)])

    return mf(user_ids, item_ids, utab2, itab2)


# speedup vs baseline: 1.8669x; 1.8669x over previous
import functools

import jax
import jax.numpy as jnp
from jax import lax
from jax.experimental import pallas as pl
from jax.experimental.pallas import tpu as pltpu
from jax.experimental.pallas import tpu_sc as plsc

_LANES = 16
_CHUNK = 32


def kernel(user_ids, item_ids, user_table, item_table):
    batch = user_ids.shape[0]
    dim = user_table.shape[1]
    info = plsc.get_sparse_core_info()
    num_cores, num_subcores = info.num_cores, info.num_subcores
    num_workers = num_cores * num_subcores
    bpw = batch // num_workers
    nch = bpw // _CHUNK
    assert nch % 2 == 0

    mesh = plsc.VectorSubcoreMesh(core_axis_name="c", subcore_axis_name="s")

    @functools.partial(
        pl.kernel,
        out_type=jax.ShapeDtypeStruct((batch,), jnp.float32),
        mesh=mesh,
        scratch_types=[
            pltpu.VMEM((bpw,), jnp.int32),
            pltpu.VMEM((bpw,), jnp.int32),
            pltpu.VMEM((2, _CHUNK, 64), jnp.float32),
            pltpu.VMEM((2, _CHUNK, 64), jnp.float32),
            pltpu.VMEM((bpw,), jnp.float32),
            pltpu.VMEM((_LANES * (_LANES + 1),), jnp.float32),
            pltpu.SemaphoreType.DMA,
            pltpu.SemaphoreType.DMA,
        ],
        compiler_params=pltpu.CompilerParams(needs_layout_passes=False),
    )
    def mf(uids_hbm, iids_hbm, utab_hbm, itab_hbm, out_hbm,
           uidx_v, iidx_v, urows_v, irows_v, out_v, tr_v, sems0, sems1):
        sems = (sems0, sems1)
        wid = lax.axis_index("s") * num_cores + lax.axis_index("c")
        base = wid * bpw
        pltpu.sync_copy(uids_hbm.at[pl.ds(base, bpw)], uidx_v)
        pltpu.sync_copy(iids_hbm.at[pl.ds(base, bpw)], iidx_v)

        def fire(c, buf, sem):
            for g in range(_CHUNK // _LANES):
                uvec = uidx_v[pl.ds(c * _CHUNK + g * _LANES, _LANES)]
                ivec = iidx_v[pl.ds(c * _CHUNK + g * _LANES, _LANES)]
                for rr in range(_LANES):
                    j = g * _LANES + rr
                    pltpu.async_copy(
                        utab_hbm.at[uvec[rr]], urows_v.at[buf, j], sem)
                    pltpu.async_copy(
                        itab_hbm.at[ivec[rr]], irows_v.at[buf, j], sem)

        def wait_chunk(sem):
            pltpu.make_async_copy(
                utab_hbm.at[pl.ds(0, _CHUNK)], urows_v.at[0], sem).wait()
            pltpu.make_async_copy(
                itab_hbm.at[pl.ds(0, _CHUNK)], irows_v.at[0], sem).wait()

        lane_iota = lax.iota(jnp.int32, _LANES)
        tr_idx_base = lane_iota * (_LANES + 1)

        def compute(c, buf):
            for gg in range(_CHUNK // _LANES):
                for rr in range(_LANES):
                    j = gg * _LANES + rr
                    acc = None
                    for c4 in range(dim // _LANES):
                        u = urows_v[buf, j, pl.ds(c4 * _LANES, _LANES)]
                        v = irows_v[buf, j, pl.ds(c4 * _LANES, _LANES)]
                        p = u * v
                        acc = p if acc is None else acc + p
                    plsc.store_scatter(tr_v, [tr_idx_base + rr], acc)
                res = None
                for cc in range(_LANES):
                    col = tr_v[pl.ds(cc * (_LANES + 1), _LANES)]
                    res = col if res is None else res + col
                out_v[pl.ds(c * _CHUNK + gg * _LANES, _LANES)] = res

        fire(0, 0, sems[0])

        def body(c2, carry):
            c = 2 * c2
            fire(c + 1, 1, sems[1])
            wait_chunk(sems[0])
            compute(c, 0)

            @pl.when(c + 2 < nch)
            def _():
                fire(c + 2, 0, sems[0])

            wait_chunk(sems[1])
            compute(c + 1, 1)
            return carry

        lax.fori_loop(0, nch // 2, body, 0)
        pltpu.sync_copy(out_v, out_hbm.at[pl.ds(base, bpw)])

    return mf(user_ids, item_ids, user_table, item_table)
